# baseline (speedup 1.0000x reference)
import functools

import jax
import jax.numpy as jnp
from jax import lax
from jax.experimental import pallas as pl
from jax.experimental.pallas import tpu as pltpu
from jax.experimental.pallas import tpu_sc as plsc


def _sc_row_gather(table, idx, chunk=32):
    b = idx.shape[0]
    d = table.shape[1]
    info = plsc.get_sparse_core_info()
    nw = info.num_cores * info.num_subcores
    bpw = b // nw
    nchunks = bpw // chunk
    mesh = plsc.VectorSubcoreMesh(core_axis_name="c", subcore_axis_name="s")

    @functools.partial(
        pl.kernel, mesh=mesh,
        out_type=jax.ShapeDtypeStruct((b, d), table.dtype),
        scratch_types=[
            pltpu.VMEM((bpw,), jnp.int32),
            pltpu.VMEM((chunk, d), table.dtype),
            pltpu.SemaphoreType.DMA,
        ],
    )
    def k(table_hbm, idx_hbm, out_hbm, idx_v, rows_v, sem):
        wid = lax.axis_index("s") * info.num_cores + lax.axis_index("c")
        base = wid * bpw
        pltpu.sync_copy(idx_hbm.at[pl.ds(base, bpw)], idx_v)

        def body(ci, carry):
            off = pl.multiple_of(ci * chunk, chunk)
            pltpu.async_copy(
                table_hbm.at[idx_v.at[pl.ds(off, chunk)]], rows_v, sem).wait()
            pltpu.sync_copy(rows_v, out_hbm.at[pl.ds(base + off, chunk)])
            return carry

        lax.fori_loop(0, nchunks, body, 0)

    return k(table, idx)


def _sc_row_scatter(rows, idx, out_rows, chunk=32):
    b = idx.shape[0]
    d = rows.shape[1]
    info = plsc.get_sparse_core_info()
    nw = info.num_cores * info.num_subcores
    bpw = b // nw
    nchunks = bpw // chunk
    idx3 = idx.reshape(nw, nchunks, chunk)
    mesh = plsc.VectorSubcoreMesh(core_axis_name="c", subcore_axis_name="s")

    @functools.partial(
        pl.kernel, mesh=mesh,
        out_type=jax.ShapeDtypeStruct((out_rows, d), rows.dtype),
        scratch_types=[
            pltpu.VMEM((nchunks, chunk), jnp.int32),
            pltpu.VMEM((chunk, d), rows.dtype),
            pltpu.SemaphoreType.DMA,
        ],
    )
    def k(rows_hbm, idx_hbm, out_hbm, idx_v, rows_v, sem):
        wid = lax.axis_index("s") * info.num_cores + lax.axis_index("c")
        base = wid * bpw
        pltpu.sync_copy(idx_hbm.at[wid], idx_v)

        def body(ci, carry):
            off = pl.multiple_of(ci * chunk, chunk)
            pltpu.sync_copy(rows_hbm.at[pl.ds(base + off, chunk)], rows_v)
            pltpu.async_copy(rows_v, out_hbm.at[idx_v.at[ci]], sem).wait()
            return carry

        lax.fori_loop(0, nchunks, body, 0)

    return k(rows, idx3)


def _row_gather(table, idx, chunk=32):
    try:
        info = plsc.get_sparse_core_info()
        nw = info.num_cores * info.num_subcores
    except Exception:
        nw = 0
    if nw and idx.shape[0] % (nw * chunk) == 0:
        return _sc_row_gather(table, idx, chunk)
    return table[idx]


def _routing(note_type_pos, n_experts, blk, num_blocks):
    t = note_type_pos.shape[0]
    e = note_type_pos.astype(jnp.int32)
    order = jnp.argsort(e, stable=True)
    es = e[order]
    counts = jnp.bincount(e, length=n_experts).astype(jnp.int32)
    blocks_per = (counts + blk - 1) // blk
    starts_blk = jnp.concatenate(
        [jnp.zeros((1,), jnp.int32), jnp.cumsum(blocks_per)[:-1].astype(jnp.int32)])
    starts_row = starts_blk * blk
    cum_counts = jnp.concatenate(
        [jnp.zeros((1,), jnp.int32), jnp.cumsum(counts)[:-1].astype(jnp.int32)])
    rank = jnp.arange(t, dtype=jnp.int32) - cum_counts[es]
    slot_sorted = starts_row[es] + rank
    slot_tok = jnp.zeros((t,), jnp.int32).at[order].set(slot_sorted)
    p = num_blocks * blk
    src = jnp.zeros((p,), jnp.int32).at[slot_sorted].set(order)
    block_expert = jnp.clip(
        jnp.searchsorted(starts_blk, jnp.arange(num_blocks, dtype=jnp.int32),
                         side="right").astype(jnp.int32) - 1,
        0, n_experts - 1)
    return slot_tok, src, block_expert


_DOT_DIMS = (((1,), (0,)), ((), ()))


def _gmm1_body(be_ref, x_ref, w1_ref, b1_ref, h_ref):
    acc = lax.dot_general(x_ref[...], w1_ref[0], _DOT_DIMS,
                          preferred_element_type=jnp.float32)
    h_ref[...] = jnp.maximum(acc + b1_ref[0], 0.0).astype(h_ref.dtype)


def _gmm2_body(be_ref, h_ref, w2_ref, b2_ref, o_ref):
    acc = lax.dot_general(h_ref[...], w2_ref[0], _DOT_DIMS,
                          preferred_element_type=jnp.float32)
    o_ref[...] = acc + b2_ref[0]


def _b3d(b):
    return b[:, None, :]


def kernel(x, note_type_pos, W1, b1, W2, b2):
    t, h_dim = x.shape
    n, _, ff = W1.shape
    blk = 256 if t >= 256 else 8
    fft = 2048 if ff >= 2048 else ff
    ht = 512 if h_dim >= 512 else h_dim
    num_blocks = (t + n * blk) // blk
    p = num_blocks * blk

    slot_tok, src, block_expert = _routing(note_type_pos, n, blk, num_blocks)

    x_bf = x.astype(jnp.bfloat16)
    try:
        info = plsc.get_sparse_core_info()
        nw = info.num_cores * info.num_subcores
    except Exception:
        nw = 0
    if nw and t % (nw * 32) == 0 and h_dim % 2 == 0:
        x_pk = lax.bitcast_convert_type(
            x_bf.reshape(t, h_dim // 2, 2), jnp.float32)
        xs_pk = _sc_row_scatter(x_pk, slot_tok, p)
        xs = lax.bitcast_convert_type(xs_pk, jnp.bfloat16).reshape(p, h_dim)
    else:
        xs = x_bf[src]

    nj1 = ff // fft
    hs = pl.pallas_call(
        _gmm1_body,
        grid_spec=pltpu.PrefetchScalarGridSpec(
            num_scalar_prefetch=1,
            grid=(nj1, num_blocks),
            in_specs=[
                pl.BlockSpec((blk, h_dim), lambda jf, i, be: (i, 0)),
                pl.BlockSpec((1, h_dim, fft), lambda jf, i, be: (be[i], 0, jf)),
                pl.BlockSpec((1, 1, fft), lambda jf, i, be: (be[i], 0, jf)),
            ],
            out_specs=pl.BlockSpec((blk, fft), lambda jf, i, be: (i, jf)),
        ),
        out_shape=jax.ShapeDtypeStruct((p, ff), jnp.bfloat16),
    )(block_expert, xs, W1, _b3d(b1))

    nj2 = h_dim // ht
    outs = pl.pallas_call(
        _gmm2_body,
        grid_spec=pltpu.PrefetchScalarGridSpec(
            num_scalar_prefetch=1,
            grid=(nj2, num_blocks),
            in_specs=[
                pl.BlockSpec((blk, ff), lambda jh, i, be: (i, 0)),
                pl.BlockSpec((1, ff, ht), lambda jh, i, be: (be[i], 0, jh)),
                pl.BlockSpec((1, 1, ht), lambda jh, i, be: (be[i], 0, jh)),
            ],
            out_specs=pl.BlockSpec((blk, ht), lambda jh, i, be: (i, jh)),
        ),
        out_shape=jax.ShapeDtypeStruct((p, h_dim), jnp.float32),
    )(block_expert, hs, W2, _b3d(b2))

    return _row_gather(outs, slot_tok)


# speedup vs baseline: 1.5185x; 1.5185x over previous
import functools

import jax
import jax.numpy as jnp
from jax import lax
from jax.experimental import pallas as pl
from jax.experimental.pallas import tpu as pltpu
from jax.experimental.pallas import tpu_sc as plsc


def _sc_row_gather(table, idx, chunk=32):
    b = idx.shape[0]
    d = table.shape[1]
    info = plsc.get_sparse_core_info()
    nw = info.num_cores * info.num_subcores
    bpw = b // nw
    nchunks = bpw // chunk
    mesh = plsc.VectorSubcoreMesh(core_axis_name="c", subcore_axis_name="s")

    @functools.partial(
        pl.kernel, mesh=mesh,
        out_type=jax.ShapeDtypeStruct((b, d), table.dtype),
        scratch_types=[
            pltpu.VMEM((bpw,), jnp.int32),
            pltpu.VMEM((chunk, d), table.dtype),
            pltpu.SemaphoreType.DMA,
        ],
    )
    def k(table_hbm, idx_hbm, out_hbm, idx_v, rows_v, sem):
        wid = lax.axis_index("s") * info.num_cores + lax.axis_index("c")
        base = wid * bpw
        pltpu.sync_copy(idx_hbm.at[pl.ds(base, bpw)], idx_v)

        def body(ci, carry):
            off = pl.multiple_of(ci * chunk, chunk)
            pltpu.async_copy(
                table_hbm.at[idx_v.at[pl.ds(off, chunk)]], rows_v, sem).wait()
            pltpu.sync_copy(rows_v, out_hbm.at[pl.ds(base + off, chunk)])
            return carry

        lax.fori_loop(0, nchunks, body, 0)

    return k(table, idx)


def _sc_row_scatter(rows, idx, out_rows, chunk=32):
    b = idx.shape[0]
    d = rows.shape[1]
    info = plsc.get_sparse_core_info()
    nw = info.num_cores * info.num_subcores
    bpw = b // nw
    nchunks = bpw // chunk
    idx3 = idx.reshape(nw, nchunks, chunk)
    mesh = plsc.VectorSubcoreMesh(core_axis_name="c", subcore_axis_name="s")

    @functools.partial(
        pl.kernel, mesh=mesh,
        out_type=jax.ShapeDtypeStruct((out_rows, d), rows.dtype),
        scratch_types=[
            pltpu.VMEM((nchunks, chunk), jnp.int32),
            pltpu.VMEM((chunk, d), rows.dtype),
            pltpu.SemaphoreType.DMA,
        ],
    )
    def k(rows_hbm, idx_hbm, out_hbm, idx_v, rows_v, sem):
        wid = lax.axis_index("s") * info.num_cores + lax.axis_index("c")
        base = wid * bpw
        pltpu.sync_copy(idx_hbm.at[wid], idx_v)

        def body(ci, carry):
            off = pl.multiple_of(ci * chunk, chunk)
            pltpu.sync_copy(rows_hbm.at[pl.ds(base + off, chunk)], rows_v)
            pltpu.async_copy(rows_v, out_hbm.at[idx_v.at[ci]], sem).wait()
            return carry

        lax.fori_loop(0, nchunks, body, 0)

    return k(rows, idx3)


def _row_gather(table, idx, chunk=32):
    try:
        info = plsc.get_sparse_core_info()
        nw = info.num_cores * info.num_subcores
    except Exception:
        nw = 0
    if nw and idx.shape[0] % (nw * chunk) == 0:
        return _sc_row_gather(table, idx, chunk)
    return table[idx]


def _routing(note_type_pos, n_experts, blk, num_blocks):
    t = note_type_pos.shape[0]
    e = note_type_pos.astype(jnp.int32)
    order = jnp.argsort(e, stable=True)
    es = e[order]
    counts = jnp.bincount(e, length=n_experts).astype(jnp.int32)
    blocks_per = (counts + blk - 1) // blk
    starts_blk = jnp.concatenate(
        [jnp.zeros((1,), jnp.int32), jnp.cumsum(blocks_per)[:-1].astype(jnp.int32)])
    starts_row = starts_blk * blk
    cum_counts = jnp.concatenate(
        [jnp.zeros((1,), jnp.int32), jnp.cumsum(counts)[:-1].astype(jnp.int32)])
    rank = jnp.arange(t, dtype=jnp.int32) - cum_counts[es]
    slot_sorted = starts_row[es] + rank
    slot_tok = jnp.zeros((t,), jnp.int32).at[order].set(slot_sorted)
    p = num_blocks * blk
    src = jnp.zeros((p,), jnp.int32).at[slot_sorted].set(order)
    block_expert = jnp.clip(
        jnp.searchsorted(starts_blk, jnp.arange(num_blocks, dtype=jnp.int32),
                         side="right").astype(jnp.int32) - 1,
        0, n_experts - 1)
    return slot_tok, src, block_expert


_DOT_DIMS = (((1,), (0,)), ((), ()))


def _gmm1_body(be_ref, x_ref, w1_ref, b1_ref, h_ref):
    acc = lax.dot_general(x_ref[...], w1_ref[0], _DOT_DIMS,
                          preferred_element_type=jnp.float32)
    h_ref[...] = jnp.maximum(acc + b1_ref[0], 0.0).astype(h_ref.dtype)


def _gmm2_body(be_ref, h_ref, w2_ref, b2_ref, o_ref):
    acc = lax.dot_general(h_ref[...], w2_ref[0], _DOT_DIMS,
                          preferred_element_type=jnp.float32)
    o_ref[...] = acc + b2_ref[0]


def _b3d(b):
    return b[:, None, :]


def kernel(x, note_type_pos, W1, b1, W2, b2):
    t, h_dim = x.shape
    n, _, ff = W1.shape
    blk = 256 if t >= 256 else 8
    fft = 2048 if ff >= 2048 else ff
    ht = 512 if h_dim >= 512 else h_dim
    num_blocks = (t + n * blk) // blk
    p = num_blocks * blk

    slot_tok, src, block_expert = _routing(note_type_pos, n, blk, num_blocks)

    try:
        info = plsc.get_sparse_core_info()
        nw = info.num_cores * info.num_subcores
    except Exception:
        nw = 0
    if nw and t % (nw * 32) == 0:
        xs = _sc_row_scatter(x, slot_tok, p)
    else:
        xs = x[src]

    nj1 = ff // fft
    hs = pl.pallas_call(
        _gmm1_body,
        grid_spec=pltpu.PrefetchScalarGridSpec(
            num_scalar_prefetch=1,
            grid=(nj1, num_blocks),
            in_specs=[
                pl.BlockSpec((blk, h_dim), lambda jf, i, be: (i, 0)),
                pl.BlockSpec((1, h_dim, fft), lambda jf, i, be: (be[i], 0, jf)),
                pl.BlockSpec((1, 1, fft), lambda jf, i, be: (be[i], 0, jf)),
            ],
            out_specs=pl.BlockSpec((blk, fft), lambda jf, i, be: (i, jf)),
        ),
        out_shape=jax.ShapeDtypeStruct((p, ff), jnp.bfloat16),
    )(block_expert, xs, W1, _b3d(b1))

    nj2 = h_dim // ht
    outs = pl.pallas_call(
        _gmm2_body,
        grid_spec=pltpu.PrefetchScalarGridSpec(
            num_scalar_prefetch=1,
            grid=(nj2, num_blocks),
            in_specs=[
                pl.BlockSpec((blk, ff), lambda jh, i, be: (i, 0)),
                pl.BlockSpec((1, ff, ht), lambda jh, i, be: (be[i], 0, jh)),
                pl.BlockSpec((1, 1, ht), lambda jh, i, be: (be[i], 0, jh)),
            ],
            out_specs=pl.BlockSpec((blk, ht), lambda jh, i, be: (i, jh)),
        ),
        out_shape=jax.ShapeDtypeStruct((p, h_dim), jnp.float32),
    )(block_expert, hs, W2, _b3d(b2))

    return _row_gather(outs, slot_tok)


# speedup vs baseline: 1.5832x; 1.0426x over previous
import functools

import jax
import jax.numpy as jnp
from jax import lax
from jax.experimental import pallas as pl
from jax.experimental.pallas import tpu as pltpu
from jax.experimental.pallas import tpu_sc as plsc


def _sc_row_gather(table, idx, chunk=32):
    b = idx.shape[0]
    d = table.shape[1]
    info = plsc.get_sparse_core_info()
    nw = info.num_cores * info.num_subcores
    bpw = b // nw
    nchunks = bpw // chunk
    mesh = plsc.VectorSubcoreMesh(core_axis_name="c", subcore_axis_name="s")

    @functools.partial(
        pl.kernel, mesh=mesh,
        out_type=jax.ShapeDtypeStruct((b, d), table.dtype),
        scratch_types=[
            pltpu.VMEM((bpw,), jnp.int32),
            pltpu.VMEM((chunk, d), table.dtype),
            pltpu.SemaphoreType.DMA,
        ],
    )
    def k(table_hbm, idx_hbm, out_hbm, idx_v, rows_v, sem):
        wid = lax.axis_index("s") * info.num_cores + lax.axis_index("c")
        base = wid * bpw
        pltpu.sync_copy(idx_hbm.at[pl.ds(base, bpw)], idx_v)

        def body(ci, carry):
            off = pl.multiple_of(ci * chunk, chunk)
            pltpu.async_copy(
                table_hbm.at[idx_v.at[pl.ds(off, chunk)]], rows_v, sem).wait()
            pltpu.sync_copy(rows_v, out_hbm.at[pl.ds(base + off, chunk)])
            return carry

        lax.fori_loop(0, nchunks, body, 0)

    return k(table, idx)


def _sc_row_scatter(rows, idx, out_rows, chunk=32):
    b = idx.shape[0]
    d = rows.shape[1]
    info = plsc.get_sparse_core_info()
    nw = info.num_cores * info.num_subcores
    bpw = b // nw
    nchunks = bpw // chunk
    idx3 = idx.reshape(nw, nchunks, chunk)
    mesh = plsc.VectorSubcoreMesh(core_axis_name="c", subcore_axis_name="s")

    @functools.partial(
        pl.kernel, mesh=mesh,
        out_type=jax.ShapeDtypeStruct((out_rows, d), rows.dtype),
        scratch_types=[
            pltpu.VMEM((nchunks, chunk), jnp.int32),
            pltpu.VMEM((chunk, d), rows.dtype),
            pltpu.SemaphoreType.DMA,
        ],
    )
    def k(rows_hbm, idx_hbm, out_hbm, idx_v, rows_v, sem):
        wid = lax.axis_index("s") * info.num_cores + lax.axis_index("c")
        base = wid * bpw
        pltpu.sync_copy(idx_hbm.at[wid], idx_v)

        def body(ci, carry):
            off = pl.multiple_of(ci * chunk, chunk)
            pltpu.sync_copy(rows_hbm.at[pl.ds(base + off, chunk)], rows_v)
            pltpu.async_copy(rows_v, out_hbm.at[idx_v.at[ci]], sem).wait()
            return carry

        lax.fori_loop(0, nchunks, body, 0)

    return k(rows, idx3)


def _row_gather(table, idx, chunk=32):
    try:
        info = plsc.get_sparse_core_info()
        nw = info.num_cores * info.num_subcores
    except Exception:
        nw = 0
    if nw and idx.shape[0] % (nw * chunk) == 0:
        return _sc_row_gather(table, idx, chunk)
    return table[idx]


def _routing(note_type_pos, n_experts, blk, num_blocks):
    e = note_type_pos.astype(jnp.int32)
    onehot = (e[:, None] == jnp.arange(n_experts, dtype=jnp.int32)[None, :]
              ).astype(jnp.int32)
    csum = jnp.cumsum(onehot, axis=0)
    counts = csum[-1]
    rank = jnp.sum(csum * onehot, axis=1) - 1
    blocks_per = (counts + blk - 1) // blk
    starts_blk = jnp.concatenate(
        [jnp.zeros((1,), jnp.int32),
         jnp.cumsum(blocks_per)[:-1].astype(jnp.int32)])
    starts_row = starts_blk * blk
    slot_tok = jnp.sum(starts_row[None, :] * onehot, axis=1) + rank
    bids = jnp.arange(num_blocks, dtype=jnp.int32)
    block_expert = jnp.clip(
        jnp.sum((bids[:, None] >= starts_blk[None, :]).astype(jnp.int32),
                axis=1) - 1,
        0, n_experts - 1)
    return slot_tok, block_expert


_DOT_DIMS = (((1,), (0,)), ((), ()))


def _gmm1_body(be_ref, x_ref, w1_ref, b1_ref, h_ref):
    acc = lax.dot_general(x_ref[...], w1_ref[0], _DOT_DIMS,
                          preferred_element_type=jnp.float32)
    h_ref[...] = jnp.maximum(acc + b1_ref[0], 0.0).astype(h_ref.dtype)


def _gmm2_body(be_ref, h_ref, w2_ref, b2_ref, o_ref):
    acc = lax.dot_general(h_ref[...], w2_ref[0], _DOT_DIMS,
                          preferred_element_type=jnp.float32)
    o_ref[...] = acc + b2_ref[0]


def _b3d(b):
    return b[:, None, :]


def kernel(x, note_type_pos, W1, b1, W2, b2):
    t, h_dim = x.shape
    n, _, ff = W1.shape
    blk = 256 if t >= 256 else 8
    fft = 2048 if ff >= 2048 else ff
    ht = 512 if h_dim >= 512 else h_dim
    num_blocks = (t + n * blk) // blk
    p = num_blocks * blk

    slot_tok, block_expert = _routing(note_type_pos, n, blk, num_blocks)

    try:
        info = plsc.get_sparse_core_info()
        nw = info.num_cores * info.num_subcores
    except Exception:
        nw = 0
    if nw and t % (nw * 32) == 0:
        xs = _sc_row_scatter(x, slot_tok, p)
    else:
        src = jnp.zeros((p,), jnp.int32).at[slot_tok].set(
            jnp.arange(t, dtype=jnp.int32))
        xs = x[src]

    nj1 = ff // fft
    hs = pl.pallas_call(
        _gmm1_body,
        grid_spec=pltpu.PrefetchScalarGridSpec(
            num_scalar_prefetch=1,
            grid=(nj1, num_blocks),
            in_specs=[
                pl.BlockSpec((blk, h_dim), lambda jf, i, be: (i, 0)),
                pl.BlockSpec((1, h_dim, fft), lambda jf, i, be: (be[i], 0, jf)),
                pl.BlockSpec((1, 1, fft), lambda jf, i, be: (be[i], 0, jf)),
            ],
            out_specs=pl.BlockSpec((blk, fft), lambda jf, i, be: (i, jf)),
        ),
        out_shape=jax.ShapeDtypeStruct((p, ff), jnp.bfloat16),
    )(block_expert, xs, W1, _b3d(b1))

    nj2 = h_dim // ht
    outs = pl.pallas_call(
        _gmm2_body,
        grid_spec=pltpu.PrefetchScalarGridSpec(
            num_scalar_prefetch=1,
            grid=(nj2, num_blocks),
            in_specs=[
                pl.BlockSpec((blk, ff), lambda jh, i, be: (i, 0)),
                pl.BlockSpec((1, ff, ht), lambda jh, i, be: (be[i], 0, jh)),
                pl.BlockSpec((1, 1, ht), lambda jh, i, be: (be[i], 0, jh)),
            ],
            out_specs=pl.BlockSpec((blk, ht), lambda jh, i, be: (i, jh)),
        ),
        out_shape=jax.ShapeDtypeStruct((p, h_dim), jnp.float32),
    )(block_expert, hs, W2, _b3d(b2))

    return _row_gather(outs, slot_tok)
